# baseline (speedup 1.0000x reference)
import jax
import jax.numpy as jnp
from jax.experimental import pallas as pl
from jax.experimental.pallas import tpu as pltpu

_D_MODEL = 4096
_N_EXPERTS = 64
_GROUP = 4
_MARGIN = 0.1
_BOOST_EPS = 0.0001


def _router_kernel(x_ref, w_ref, b_ref, o_ref):
    x = x_ref[...]
    logits = jax.lax.dot_general(
        x,
        w_ref[...],
        dimension_numbers=(((1,), (1,)), ((), ())),
        preferred_element_type=jnp.float32,
    )
    logits = logits + b_ref[...]

    bt = logits.shape[0]
    lane = jax.lax.broadcasted_iota(jnp.int32, (bt, _N_EXPERTS), 1)
    even = (lane & 1) == 0
    low2 = (lane & 2) == 0

    y = jnp.maximum(
        logits,
        jnp.where(even, pltpu.roll(logits, 63, 1), pltpu.roll(logits, 1, 1)),
    )
    mx = jnp.maximum(
        y, jnp.where(low2, pltpu.roll(y, 62, 1), pltpu.roll(y, 2, 1))
    )

    w = ((mx - logits) < _MARGIN).astype(jnp.float32)
    c = w + jnp.where(even, pltpu.roll(w, 63, 1), pltpu.roll(w, 1, 1))
    cnt = c + jnp.where(low2, pltpu.roll(c, 62, 1), pltpu.roll(c, 2, 1))

    overwrite = ((lane & (_GROUP - 1)) == 0) & (cnt > 1.5)
    o_ref[...] = jnp.where(overwrite, mx + _BOOST_EPS, logits)


def kernel(hidden_states, W, b):
    T, D = hidden_states.shape
    BT = 1024
    b2 = b.reshape(1, _N_EXPERTS)
    return pl.pallas_call(
        _router_kernel,
        grid=(T // BT,),
        in_specs=[
            pl.BlockSpec((BT, D), lambda i: (i, 0)),
            pl.BlockSpec((_N_EXPERTS, D), lambda i: (0, 0)),
            pl.BlockSpec((1, _N_EXPERTS), lambda i: (0, 0)),
        ],
        out_specs=pl.BlockSpec((BT, _N_EXPERTS), lambda i: (i, 0)),
        out_shape=jax.ShapeDtypeStruct((T, _N_EXPERTS), jnp.float32),
        compiler_params=pltpu.CompilerParams(
            dimension_semantics=("arbitrary",),
        ),
    )(hidden_states, W, b2)


# speedup vs baseline: 1.0043x; 1.0043x over previous
import jax
import jax.numpy as jnp
from jax.experimental import pallas as pl
from jax.experimental.pallas import tpu as pltpu

_D_MODEL = 4096
_N_EXPERTS = 64
_GROUP = 4
_MARGIN = 0.1
_BOOST_EPS = 0.0001


_RC = 256


def _router_kernel(x_ref, w_ref, b_ref, o_ref):
    w_mat = w_ref[...]
    bias = b_ref[...]
    lane = jax.lax.broadcasted_iota(jnp.int32, (_RC, _N_EXPERTS), 1)
    even = (lane & 1) == 0
    low2 = (lane & 2) == 0
    canon = (lane & (_GROUP - 1)) == 0

    for r in range(0, x_ref.shape[0], _RC):
        logits = jax.lax.dot_general(
            x_ref[pl.ds(r, _RC), :],
            w_mat,
            dimension_numbers=(((1,), (1,)), ((), ())),
            preferred_element_type=jnp.float32,
        )
        logits = logits + bias

        y = jnp.maximum(
            logits,
            jnp.where(even, pltpu.roll(logits, 63, 1), pltpu.roll(logits, 1, 1)),
        )
        mx = jnp.maximum(
            y, jnp.where(low2, pltpu.roll(y, 62, 1), pltpu.roll(y, 2, 1))
        )

        w = ((mx - logits) < _MARGIN).astype(jnp.float32)
        c = w + jnp.where(even, pltpu.roll(w, 63, 1), pltpu.roll(w, 1, 1))
        cnt = c + jnp.where(low2, pltpu.roll(c, 62, 1), pltpu.roll(c, 2, 1))

        overwrite = canon & (cnt > 1.5)
        o_ref[pl.ds(r, _RC), :] = jnp.where(overwrite, mx + _BOOST_EPS, logits)


def kernel(hidden_states, W, b):
    T, D = hidden_states.shape
    BT = 1024
    b2 = b.reshape(1, _N_EXPERTS)
    return pl.pallas_call(
        _router_kernel,
        grid=(T // BT,),
        in_specs=[
            pl.BlockSpec((BT, D), lambda i: (i, 0)),
            pl.BlockSpec((_N_EXPERTS, D), lambda i: (0, 0)),
            pl.BlockSpec((1, _N_EXPERTS), lambda i: (0, 0)),
        ],
        out_specs=pl.BlockSpec((BT, _N_EXPERTS), lambda i: (i, 0)),
        out_shape=jax.ShapeDtypeStruct((T, _N_EXPERTS), jnp.float32),
        compiler_params=pltpu.CompilerParams(
            dimension_semantics=("arbitrary",),
        ),
    )(hidden_states, W, b2)
